# baseline (speedup 1.0000x reference)
import functools

import jax
import jax.numpy as jnp
from jax.experimental import pallas as pl
from jax.experimental.pallas import tpu as pltpu

EBLK = 12800
NBLK = 10000


def _tables_body(x_ref, wp_ref, wq_ref, p_ref, q_ref):
    x = x_ref[...]
    p_ref[...] = jnp.dot(x, wp_ref[...], preferred_element_type=jnp.float32)
    q_ref[...] = jnp.dot(x, wq_ref[...], preferred_element_type=jnp.float32)


def _node_tables(x, wp, wq):
    n, d = x.shape
    grid = (n // NBLK,)
    return pl.pallas_call(
        _tables_body,
        grid=grid,
        in_specs=[
            pl.BlockSpec((NBLK, d), lambda b: (b, 0)),
            pl.BlockSpec((d, 16), lambda b: (0, 0)),
            pl.BlockSpec((d, 16), lambda b: (0, 0)),
        ],
        out_specs=[
            pl.BlockSpec((NBLK, 16), lambda b: (b, 0)),
            pl.BlockSpec((NBLK, 16), lambda b: (b, 0)),
        ],
        out_shape=[
            jax.ShapeDtypeStruct((n, 16), jnp.float32),
            jax.ShapeDtypeStruct((n, 16), jnp.float32),
        ],
    )(x, wp, wq)


def _edge_pre_body(ea_ref, wa_ref, ba_ref, wb_ref, bb_ref, r1_ref, r2_ref):
    ea = ea_ref[...]
    r1_ref[...] = jnp.dot(ea, wa_ref[...], preferred_element_type=jnp.float32) + ba_ref[...]
    r2_ref[...] = jnp.dot(ea, wb_ref[...], preferred_element_type=jnp.float32) + bb_ref[...]


def _edge_pre(edge_attr, wa, ba, wb, bb):
    e = edge_attr.shape[0]
    grid = (e // EBLK,)
    return pl.pallas_call(
        _edge_pre_body,
        grid=grid,
        in_specs=[
            pl.BlockSpec((EBLK, 8), lambda b: (b, 0)),
            pl.BlockSpec((8, 16), lambda b: (0, 0)),
            pl.BlockSpec((1, 16), lambda b: (0, 0)),
            pl.BlockSpec((8, 16), lambda b: (0, 0)),
            pl.BlockSpec((1, 16), lambda b: (0, 0)),
        ],
        out_specs=[
            pl.BlockSpec((EBLK, 16), lambda b: (b, 0)),
            pl.BlockSpec((EBLK, 16), lambda b: (b, 0)),
        ],
        out_shape=[
            jax.ShapeDtypeStruct((e, 16), jnp.float32),
            jax.ShapeDtypeStruct((e, 16), jnp.float32),
        ],
    )(edge_attr, wa, ba, wb, bb)


def _edge_mlp_body(pi_ref, qj_ref, r_ref, w2_ref, b2_ref, out_ref):
    h1 = jnp.maximum(pi_ref[...] + qj_ref[...] + r_ref[...], 0.0)
    h2 = jnp.dot(h1, w2_ref[...], preferred_element_type=jnp.float32) + b2_ref[...]
    out_ref[...] = jnp.maximum(h2, 0.0)


def _edge_mlp(pi, qj, r, w2, b2):
    e = pi.shape[0]
    grid = (e // EBLK,)
    return pl.pallas_call(
        _edge_mlp_body,
        grid=grid,
        in_specs=[
            pl.BlockSpec((EBLK, 16), lambda b: (b, 0)),
            pl.BlockSpec((EBLK, 16), lambda b: (b, 0)),
            pl.BlockSpec((EBLK, 16), lambda b: (b, 0)),
            pl.BlockSpec((16, 16), lambda b: (0, 0)),
            pl.BlockSpec((1, 16), lambda b: (0, 0)),
        ],
        out_specs=pl.BlockSpec((EBLK, 16), lambda b: (b, 0)),
        out_shape=jax.ShapeDtypeStruct((e, 16), jnp.float32),
    )(pi, qj, r, w2, b2)


def _head_body(x_ref, w1_ref, b1_ref, w2_ref, b2_ref, out_ref):
    h = jnp.maximum(jnp.dot(x_ref[...], w1_ref[...], preferred_element_type=jnp.float32) + b1_ref[...], 0.0)
    out_ref[...] = jnp.dot(h, w2_ref[...], preferred_element_type=jnp.float32) + b2_ref[...]


def _head(x2, wl1, bl1, wl2, bl2):
    n = x2.shape[0]
    grid = (n // NBLK,)
    return pl.pallas_call(
        _head_body,
        grid=grid,
        in_specs=[
            pl.BlockSpec((NBLK, 16), lambda b: (b, 0)),
            pl.BlockSpec((16, 16), lambda b: (0, 0)),
            pl.BlockSpec((1, 16), lambda b: (0, 0)),
            pl.BlockSpec((16, 1), lambda b: (0, 0)),
            pl.BlockSpec((1, 1), lambda b: (0, 0)),
        ],
        out_specs=pl.BlockSpec((NBLK, 1), lambda b: (b, 0)),
        out_shape=jax.ShapeDtypeStruct((n, 1), jnp.float32),
    )(x2, wl1, bl1, wl2, bl2)


def kernel(x, edge_index, edge_attr, W1a, b1a, W2a, b2a, W1b, b1b, W2b, b2b, Wl1, bl1, Wl2, bl2):
    n = x.shape[0]
    i = edge_index[0]
    j = edge_index[1]

    r1, r2 = _edge_pre(
        edge_attr,
        W1a[:, 8:16].T, b1a.reshape(1, 16),
        W1b[:, 32:40].T, b1b.reshape(1, 16),
    )

    p1, q1 = _node_tables(x, W1a[:, 0:4].T, W1a[:, 4:8].T)
    h2 = _edge_mlp(p1[i], q1[j], r1, W2a.T, b2a.reshape(1, 16))
    x1 = jax.ops.segment_max(h2, i, num_segments=n)
    x1 = jnp.where(jnp.isneginf(x1), 0.0, x1)

    p2, q2 = _node_tables(x1, W1b[:, 0:16].T, W1b[:, 16:32].T)
    h2b = _edge_mlp(p2[i], q2[j], r2, W2b.T, b2b.reshape(1, 16))
    x2 = jax.ops.segment_max(h2b, i, num_segments=n)
    x2 = jnp.where(jnp.isneginf(x2), 0.0, x2)

    return _head(x2, Wl1.T, bl1.reshape(1, 16), Wl2.T, bl2.reshape(1, 1))


# speedup vs baseline: 3.7040x; 3.7040x over previous
import functools

import jax
import jax.numpy as jnp
from jax import lax
from jax.experimental import pallas as pl
from jax.experimental.pallas import tpu as pltpu
from jax.experimental.pallas import tpu_sc as plsc

EBLK = 12800
NBLK = 10000


def _tables_body(x_ref, wp_ref, wq_ref, p_ref, q_ref):
    x = x_ref[...]
    p_ref[...] = jnp.dot(x, wp_ref[...], preferred_element_type=jnp.float32)
    q_ref[...] = jnp.dot(x, wq_ref[...], preferred_element_type=jnp.float32)


def _node_tables(x, wp, wq):
    n, d = x.shape
    grid = (n // NBLK,)
    return pl.pallas_call(
        _tables_body,
        grid=grid,
        in_specs=[
            pl.BlockSpec((NBLK, d), lambda b: (b, 0)),
            pl.BlockSpec((d, 16), lambda b: (0, 0)),
            pl.BlockSpec((d, 16), lambda b: (0, 0)),
        ],
        out_specs=[
            pl.BlockSpec((NBLK, 16), lambda b: (b, 0)),
            pl.BlockSpec((NBLK, 16), lambda b: (b, 0)),
        ],
        out_shape=[
            jax.ShapeDtypeStruct((n, 16), jnp.float32),
            jax.ShapeDtypeStruct((n, 16), jnp.float32),
        ],
    )(x, wp, wq)


def _tables_merge_body(xa_ref, xb_ref, wp_ref, wq_ref, p_ref, q_ref, x_ref):
    x = jnp.maximum(xa_ref[...], xb_ref[...])
    x_ref[...] = x
    p_ref[...] = jnp.dot(x, wp_ref[...], preferred_element_type=jnp.float32)
    q_ref[...] = jnp.dot(x, wq_ref[...], preferred_element_type=jnp.float32)


def _node_tables_merge(xa, xb, wp, wq):
    n, d = xa.shape
    grid = (n // NBLK,)
    return pl.pallas_call(
        _tables_merge_body,
        grid=grid,
        in_specs=[
            pl.BlockSpec((NBLK, d), lambda b: (b, 0)),
            pl.BlockSpec((NBLK, d), lambda b: (b, 0)),
            pl.BlockSpec((d, 16), lambda b: (0, 0)),
            pl.BlockSpec((d, 16), lambda b: (0, 0)),
        ],
        out_specs=[
            pl.BlockSpec((NBLK, 16), lambda b: (b, 0)),
            pl.BlockSpec((NBLK, 16), lambda b: (b, 0)),
            pl.BlockSpec((NBLK, d), lambda b: (b, 0)),
        ],
        out_shape=[
            jax.ShapeDtypeStruct((n, 16), jnp.float32),
            jax.ShapeDtypeStruct((n, 16), jnp.float32),
            jax.ShapeDtypeStruct((n, d), jnp.float32),
        ],
    )(xa, xb, wp, wq)


def _edge_pre_body(ea_ref, wa_ref, ba_ref, wb_ref, bb_ref, r1_ref, r2_ref):
    ea = ea_ref[...]
    r1_ref[...] = jnp.dot(ea, wa_ref[...], preferred_element_type=jnp.float32) + ba_ref[...]
    r2_ref[...] = jnp.dot(ea, wb_ref[...], preferred_element_type=jnp.float32) + bb_ref[...]


def _edge_pre(edge_attr, wa, ba, wb, bb):
    e = edge_attr.shape[0]
    grid = (e // EBLK,)
    return pl.pallas_call(
        _edge_pre_body,
        grid=grid,
        in_specs=[
            pl.BlockSpec((EBLK, 8), lambda b: (b, 0)),
            pl.BlockSpec((8, 16), lambda b: (0, 0)),
            pl.BlockSpec((1, 16), lambda b: (0, 0)),
            pl.BlockSpec((8, 16), lambda b: (0, 0)),
            pl.BlockSpec((1, 16), lambda b: (0, 0)),
        ],
        out_specs=[
            pl.BlockSpec((EBLK, 16), lambda b: (b, 0)),
            pl.BlockSpec((EBLK, 16), lambda b: (b, 0)),
        ],
        out_shape=[
            jax.ShapeDtypeStruct((e, 16), jnp.float32),
            jax.ShapeDtypeStruct((e, 16), jnp.float32),
        ],
    )(edge_attr, wa, ba, wb, bb)


def _edge_mlp_body(pi_ref, qj_ref, r_ref, w2_ref, b2_ref, out_ref):
    h1 = jnp.maximum(pi_ref[...] + qj_ref[...] + r_ref[...], 0.0)
    h2 = jnp.dot(h1, w2_ref[...], preferred_element_type=jnp.float32) + b2_ref[...]
    out_ref[...] = jnp.maximum(h2, 0.0)


def _edge_mlp(pi, qj, r, w2, b2):
    e = pi.shape[0]
    grid = (e // EBLK,)
    return pl.pallas_call(
        _edge_mlp_body,
        grid=grid,
        in_specs=[
            pl.BlockSpec((EBLK, 16), lambda b: (b, 0)),
            pl.BlockSpec((EBLK, 16), lambda b: (b, 0)),
            pl.BlockSpec((EBLK, 16), lambda b: (b, 0)),
            pl.BlockSpec((16, 16), lambda b: (0, 0)),
            pl.BlockSpec((1, 16), lambda b: (0, 0)),
        ],
        out_specs=pl.BlockSpec((EBLK, 16), lambda b: (b, 0)),
        out_shape=jax.ShapeDtypeStruct((e, 16), jnp.float32),
    )(pi, qj, r, w2, b2)


def _head_body(xa_ref, xb_ref, w1_ref, b1_ref, w2_ref, b2_ref, out_ref):
    x = jnp.maximum(xa_ref[...], xb_ref[...])
    h = jnp.maximum(jnp.dot(x, w1_ref[...], preferred_element_type=jnp.float32) + b1_ref[...], 0.0)
    out_ref[...] = jnp.dot(h, w2_ref[...], preferred_element_type=jnp.float32) + b2_ref[...]


def _head(xa, xb, wl1, bl1, wl2, bl2):
    n = xa.shape[0]
    grid = (n // NBLK,)
    return pl.pallas_call(
        _head_body,
        grid=grid,
        in_specs=[
            pl.BlockSpec((NBLK, 16), lambda b: (b, 0)),
            pl.BlockSpec((NBLK, 16), lambda b: (b, 0)),
            pl.BlockSpec((16, 16), lambda b: (0, 0)),
            pl.BlockSpec((1, 16), lambda b: (0, 0)),
            pl.BlockSpec((16, 1), lambda b: (0, 0)),
            pl.BlockSpec((1, 1), lambda b: (0, 0)),
        ],
        out_specs=pl.BlockSpec((NBLK, 1), lambda b: (b, 0)),
        out_shape=jax.ShapeDtypeStruct((n, 1), jnp.float32),
    )(xa, xb, wl1, bl1, wl2, bl2)


_NC = 2
_NS = 16
_NW = _NC * _NS
_GCH = 2000


def _sc_gather2(p, q, i_idx, j_idx, interpret=False):
    e = i_idx.shape[0]
    per_w = e // _NW
    n_steps = per_w // _GCH
    mesh = plsc.VectorSubcoreMesh(
        core_axis_name="c", subcore_axis_name="s", num_cores=_NC, num_subcores=_NS)

    @functools.partial(
        pl.kernel,
        out_type=[
            jax.ShapeDtypeStruct((e, 16), jnp.float32),
            jax.ShapeDtypeStruct((e, 16), jnp.float32),
        ],
        mesh=mesh,
        scratch_types=[
            pltpu.VMEM((_GCH,), jnp.int32),
            pltpu.VMEM((_GCH,), jnp.int32),
            pltpu.VMEM((_GCH, 16), jnp.float32),
            pltpu.VMEM((_GCH, 16), jnp.float32),
            pltpu.SemaphoreType.DMA,
            pltpu.SemaphoreType.DMA,
        ],
        compiler_params=pltpu.CompilerParams(use_tc_tiling_on_sc=False),
        interpret=interpret,
    )
    def body(p_hbm, q_hbm, i_hbm, j_hbm, pi_hbm, qj_hbm, iv, jv, rp, rq, s1, s2):
        wid = lax.axis_index("s") * _NC + lax.axis_index("c")
        base = wid * per_w

        def step(t, carry):
            off = base + t * _GCH
            c1 = pltpu.async_copy(i_hbm.at[pl.ds(off, _GCH)], iv, s1)
            c2 = pltpu.async_copy(j_hbm.at[pl.ds(off, _GCH)], jv, s2)
            c1.wait()
            c2.wait()
            g1 = pltpu.async_copy(p_hbm.at[iv], rp, s1)
            g2 = pltpu.async_copy(q_hbm.at[jv], rq, s2)
            g1.wait()
            g2.wait()
            o1 = pltpu.async_copy(rp, pi_hbm.at[pl.ds(off, _GCH)], s1)
            o2 = pltpu.async_copy(rq, qj_hbm.at[pl.ds(off, _GCH)], s2)
            o1.wait()
            o2.wait()
            return carry

        lax.fori_loop(0, n_steps, step, 0)

    return body(p, q, i_idx, j_idx)


_SCH = 12800
_SUB = 512
_NPT = 6250


def _sc_scatter_max(h2, i_idx, n):
    e = i_idx.shape[0]
    half = e // 2
    n_chunks = half // _SCH
    n_sub = _SCH // _SUB
    mesh = plsc.VectorSubcoreMesh(
        core_axis_name="c", subcore_axis_name="s", num_cores=_NC, num_subcores=_NS)

    @functools.partial(
        pl.kernel,
        out_type=jax.ShapeDtypeStruct((2 * n * 16,), jnp.float32),
        mesh=mesh,
        scratch_types=[
            pltpu.VMEM_SHARED((_SCH, 16), jnp.float32),
            pltpu.VMEM(((_NPT + 1) * 16,), jnp.float32),
            pltpu.VMEM((2 * _SUB,), jnp.int32),
            pltpu.VMEM((1040,), jnp.int32),
            pltpu.VMEM((1040,), jnp.int32),
            pltpu.VMEM((544, 16), jnp.float32),
            pltpu.SemaphoreType.DMA,
            pltpu.SemaphoreType.DMA,
            pltpu.SemaphoreType.DMA,
            pltpu.SemaphoreType.DMA,
        ],
        compiler_params=pltpu.CompilerParams(
            needs_layout_passes=False, use_tc_tiling_on_sc=False),
    )
    def body(h_hbm, i_hbm, out_hbm, hsh, acc, iv, dbuf, pbuf, hrows,
             sem_h, sem_g, sem_a, sem_b):
        c = lax.axis_index("c")
        s = lax.axis_index("s")
        lo = s * _NPT
        hi = lo + _NPT
        lanes = lax.iota(jnp.int32, 16)
        zeros16 = jnp.zeros((16,), jnp.float32)
        pad16 = jnp.full((16,), _NPT, jnp.int32)
        zi16 = jnp.zeros((16,), jnp.int32)

        def init_acc(g, _):
            acc[pl.ds(g * 16, 16)] = zeros16
            return 0
        lax.fori_loop(0, _NPT + 1, init_acc, 0)
        for g in range(65):
            pbuf[pl.ds(g * 16, 16)] = zi16
            dbuf[pl.ds(g * 16, 16)] = pad16

        def rmw(ng):
            def g_body(g, _):
                dv = dbuf[pl.ds(g * 16, 16)]
                for k in range(16):
                    dk = dv[k]
                    a = acc[pl.ds(dk * 16, 16)]
                    h = hrows[g * 16 + k, :]
                    acc[pl.ds(dk * 16, 16)] = jnp.maximum(a, h)
                return 0
            lax.fori_loop(0, ng, g_body, 0)

        def pull(t, u, half_buf, sem):
            src = i_hbm.at[pl.ds(c * half + t * _SCH + u * _SUB, _SUB)]
            pltpu.async_copy(src, iv.at[pl.ds(half_buf * _SUB, _SUB)], sem)

        def pull_wait(t, u, half_buf, sem):
            src = i_hbm.at[pl.ds(c * half + t * _SCH + u * _SUB, _SUB)]
            pltpu.make_async_copy(
                src, iv.at[pl.ds(half_buf * _SUB, _SUB)], sem).wait()

        def scan(u, bsel, off):
            for g in range(32):
                d = iv[pl.ds(bsel * _SUB + g * 16, 16)]
                mask = (d >= lo) & (d < hi)
                slots_v = plsc.cumsum(mask.astype(jnp.int32)) + (off - 1)
                plsc.store_scatter(dbuf, [slots_v], d - lo, mask=mask)
                pv = (u * _SUB + g * 16) + lanes
                plsc.store_scatter(pbuf, [slots_v], pv, mask=mask)
                cnt = plsc.all_reduce_population_count(mask)
                off = off + cnt[0]

            @pl.when(off >= 512)
            def _():
                pltpu.async_copy(
                    hsh.at[pbuf.at[pl.ds(0, 512)]],
                    hrows.at[pl.ds(0, 512)], sem_g).wait()
                rmw(32)
                for g in range(32):
                    dbuf[pl.ds(g * 16, 16)] = dbuf[pl.ds(512 + g * 16, 16)]
                    pbuf[pl.ds(g * 16, 16)] = pbuf[pl.ds(512 + g * 16, 16)]

            return jnp.where(off >= 512, off - 512, off)

        def chunk_body(t, _):
            plsc.subcore_barrier()

            @pl.when(s == 0)
            def _():
                pltpu.async_copy(
                    h_hbm.at[pl.ds(c * half + t * _SCH, _SCH)], hsh, sem_h)
                pltpu.make_async_copy(
                    h_hbm.at[pl.ds(c * half + t * _SCH, _SCH)], hsh,
                    sem_h).wait()

            pull(t, 0, 0, sem_a)
            plsc.subcore_barrier()

            def pair_body(b, off):
                u = 2 * b
                pull_wait(t, u, 0, sem_a)
                pull(t, u + 1, 1, sem_b)
                off = scan(u, 0, off)
                pull_wait(t, u + 1, 1, sem_b)
                pull(t, u + 2, 0, sem_a)
                off = scan(u + 1, 1, off)
                return off

            off = lax.fori_loop(0, (n_sub - 1) // 2, pair_body, 0)
            pull_wait(t, n_sub - 1, 0, sem_a)
            off = scan(n_sub - 1, 0, off)

            dbuf[pl.ds(off, 16)] = pad16
            dbuf[pl.ds(off + 16, 16)] = pad16
            pbuf[pl.ds(off, 16)] = zi16
            pbuf[pl.ds(off + 16, 16)] = zi16
            ng = (off + 31) >> 4

            @pl.when(ng * 16 <= 144)
            def _():
                pltpu.async_copy(
                    hsh.at[pbuf.at[pl.ds(0, 144)]],
                    hrows.at[pl.ds(0, 144)], sem_g).wait()

            @pl.when(ng * 16 > 144)
            def _():
                pltpu.async_copy(
                    hsh.at[pbuf.at[pl.ds(0, 544)]],
                    hrows.at[pl.ds(0, 544)], sem_g).wait()

            rmw(ng)
            return 0

        lax.fori_loop(0, n_chunks, chunk_body, 0)

        obase = c * n * 16 + s * _NPT * 16
        pltpu.sync_copy(acc.at[pl.ds(0, _NPT * 16)],
                        out_hbm.at[pl.ds(obase, _NPT * 16)])

    return body(h2, i_idx)


def kernel(x, edge_index, edge_attr, W1a, b1a, W2a, b2a, W1b, b1b, W2b, b2b, Wl1, bl1, Wl2, bl2):
    n = x.shape[0]
    i = edge_index[0]
    j = edge_index[1]

    r1, r2 = _edge_pre(
        edge_attr,
        W1a[:, 8:16].T, b1a.reshape(1, 16),
        W1b[:, 32:40].T, b1b.reshape(1, 16),
    )

    p1, q1 = _node_tables(x, W1a[:, 0:4].T, W1a[:, 4:8].T)
    pi1, qj1 = _sc_gather2(p1, q1, i, j)
    h2 = _edge_mlp(pi1, qj1, r1, W2a.T, b2a.reshape(1, 16))
    part1 = _sc_scatter_max(h2, i, n).reshape(2, n, 16)

    p2, q2, _ = _node_tables_merge(part1[0], part1[1], W1b[:, 0:16].T, W1b[:, 16:32].T)
    pi2, qj2 = _sc_gather2(p2, q2, i, j)
    h2b = _edge_mlp(pi2, qj2, r2, W2b.T, b2b.reshape(1, 16))
    part2 = _sc_scatter_max(h2b, i, n).reshape(2, n, 16)

    return _head(part2[0], part2[1], Wl1.T, bl1.reshape(1, 16), Wl2.T, bl2.reshape(1, 1))


# speedup vs baseline: 8.5438x; 2.3067x over previous
import functools

import jax
import jax.numpy as jnp
from jax import lax
from jax.experimental import pallas as pl
from jax.experimental.pallas import tpu as pltpu
from jax.experimental.pallas import tpu_sc as plsc

EBLK = 12800
NBLK = 10000


EBLKP = 4000
NROWS = 12500


def _kron8(w):
    return jnp.kron(jnp.eye(8, dtype=w.dtype), w)


def _tile8(b):
    return jnp.tile(b, 8).reshape(1, -1)


def _tables_body(x_ref, wp_ref, wq_ref, p_ref, q_ref):
    x = x_ref[...]
    p_ref[...] = jnp.dot(x, wp_ref[...], preferred_element_type=jnp.float32)
    q_ref[...] = jnp.dot(x, wq_ref[...], preferred_element_type=jnp.float32)


def _node_tables(x_p, wp, wq):
    m, d = x_p.shape
    return pl.pallas_call(
        _tables_body,
        out_shape=[
            jax.ShapeDtypeStruct((m, 128), jnp.float32),
            jax.ShapeDtypeStruct((m, 128), jnp.float32),
        ],
    )(x_p, wp, wq)


def _tables_merge_body(xa_ref, xb_ref, wp_ref, wq_ref, p_ref, q_ref):
    x = jnp.maximum(xa_ref[...], xb_ref[...])
    p_ref[...] = jnp.dot(x, wp_ref[...], preferred_element_type=jnp.float32)
    q_ref[...] = jnp.dot(x, wq_ref[...], preferred_element_type=jnp.float32)


def _node_tables_merge(xa, xb, wp, wq):
    m = xa.shape[0]
    return pl.pallas_call(
        _tables_merge_body,
        out_shape=[
            jax.ShapeDtypeStruct((m, 128), jnp.float32),
            jax.ShapeDtypeStruct((m, 128), jnp.float32),
        ],
    )(xa, xb, wp, wq)


def _edge_pre_body(ea_ref, wa_ref, ba_ref, wb_ref, bb_ref, r1_ref, r2_ref):
    ea = ea_ref[...]
    r1_ref[...] = jnp.dot(ea, wa_ref[...], preferred_element_type=jnp.float32) + ba_ref[...]
    r2_ref[...] = jnp.dot(ea, wb_ref[...], preferred_element_type=jnp.float32) + bb_ref[...]


def _edge_pre(ea_p, wa, ba, wb, bb):
    m = ea_p.shape[0]
    grid = (m // EBLKP,)
    return pl.pallas_call(
        _edge_pre_body,
        grid=grid,
        in_specs=[
            pl.BlockSpec((EBLKP, 64), lambda b: (b, 0)),
            pl.BlockSpec((64, 128), lambda b: (0, 0)),
            pl.BlockSpec((1, 128), lambda b: (0, 0)),
            pl.BlockSpec((64, 128), lambda b: (0, 0)),
            pl.BlockSpec((1, 128), lambda b: (0, 0)),
        ],
        out_specs=[
            pl.BlockSpec((EBLKP, 128), lambda b: (b, 0)),
            pl.BlockSpec((EBLKP, 128), lambda b: (b, 0)),
        ],
        out_shape=[
            jax.ShapeDtypeStruct((m, 128), jnp.float32),
            jax.ShapeDtypeStruct((m, 128), jnp.float32),
        ],
    )(ea_p, wa, ba, wb, bb)


def _edge_mlp_body(pi_ref, qj_ref, r_ref, w2_ref, b2_ref, out_ref):
    h1 = jnp.maximum(pi_ref[...] + qj_ref[...] + r_ref[...], 0.0)
    h2 = jnp.dot(h1, w2_ref[...], preferred_element_type=jnp.float32) + b2_ref[...]
    out_ref[...] = jnp.maximum(h2, 0.0)


def _edge_mlp(pi, qj, r, w2, b2):
    m = pi.shape[0]
    grid = (m // EBLKP,)
    return pl.pallas_call(
        _edge_mlp_body,
        grid=grid,
        in_specs=[
            pl.BlockSpec((EBLKP, 128), lambda b: (b, 0)),
            pl.BlockSpec((EBLKP, 128), lambda b: (b, 0)),
            pl.BlockSpec((EBLKP, 128), lambda b: (b, 0)),
            pl.BlockSpec((128, 128), lambda b: (0, 0)),
            pl.BlockSpec((1, 128), lambda b: (0, 0)),
        ],
        out_specs=pl.BlockSpec((EBLKP, 128), lambda b: (b, 0)),
        out_shape=jax.ShapeDtypeStruct((m, 128), jnp.float32),
    )(pi, qj, r, w2, b2)


def _head_body(xa_ref, xb_ref, w1_ref, b1_ref, w2_ref, b2_ref, out_ref):
    x = jnp.maximum(xa_ref[...], xb_ref[...])
    h = jnp.maximum(jnp.dot(x, w1_ref[...], preferred_element_type=jnp.float32) + b1_ref[...], 0.0)
    out_ref[...] = jnp.dot(h, w2_ref[...], preferred_element_type=jnp.float32) + b2_ref[...]


def _head(xa, xb, wl1, bl1, wl2, bl2):
    m = xa.shape[0]
    return pl.pallas_call(
        _head_body,
        out_shape=jax.ShapeDtypeStruct((m, 8), jnp.float32),
    )(xa, xb, wl1, bl1, wl2, bl2)


_NC = 2
_NS = 16
_NW = _NC * _NS
_GCH = 2000

_SCH = 12800
_SUB = 512
_NPT = 6250


def _sc_scatter_max(h2, i_idx, n):
    e = i_idx.shape[0]
    half = e // 2
    n_chunks = half // _SCH
    n_sub = _SCH // _SUB
    mesh = plsc.VectorSubcoreMesh(
        core_axis_name="c", subcore_axis_name="s", num_cores=_NC, num_subcores=_NS)

    @functools.partial(
        pl.kernel,
        out_type=jax.ShapeDtypeStruct((2 * n * 16,), jnp.float32),
        mesh=mesh,
        scratch_types=[
            pltpu.VMEM_SHARED((_SCH, 16), jnp.float32),
            pltpu.VMEM(((_NPT + 1) * 16,), jnp.float32),
            pltpu.VMEM((2 * _SUB,), jnp.int32),
            pltpu.VMEM((1040,), jnp.int32),
            pltpu.VMEM((1040,), jnp.int32),
            pltpu.VMEM((544, 16), jnp.float32),
            pltpu.SemaphoreType.DMA,
            pltpu.SemaphoreType.DMA,
            pltpu.SemaphoreType.DMA,
            pltpu.SemaphoreType.DMA,
        ],
        compiler_params=pltpu.CompilerParams(
            needs_layout_passes=False, use_tc_tiling_on_sc=False),
    )
    def body(h_hbm, i_hbm, out_hbm, hsh, acc, iv, dbuf, pbuf, hrows,
             sem_h, sem_g, sem_a, sem_b):
        c = lax.axis_index("c")
        s = lax.axis_index("s")
        lo = s * _NPT
        hi = lo + _NPT
        lanes = lax.iota(jnp.int32, 16)
        zeros16 = jnp.zeros((16,), jnp.float32)
        pad16 = jnp.full((16,), _NPT, jnp.int32)
        zi16 = jnp.zeros((16,), jnp.int32)

        def init_acc(g, _):
            acc[pl.ds(g * 16, 16)] = zeros16
            return 0
        lax.fori_loop(0, _NPT + 1, init_acc, 0)
        for g in range(65):
            pbuf[pl.ds(g * 16, 16)] = zi16
            dbuf[pl.ds(g * 16, 16)] = pad16

        def rmw(ng):
            def g_body(g, _):
                dv = dbuf[pl.ds(g * 16, 16)]
                for k in range(16):
                    dk = dv[k]
                    a = acc[pl.ds(dk * 16, 16)]
                    h = hrows[g * 16 + k, :]
                    acc[pl.ds(dk * 16, 16)] = jnp.maximum(a, h)
                return 0
            lax.fori_loop(0, ng, g_body, 0)

        def pull(t, u, half_buf, sem):
            src = i_hbm.at[pl.ds(c * half + t * _SCH + u * _SUB, _SUB)]
            pltpu.async_copy(src, iv.at[pl.ds(half_buf * _SUB, _SUB)], sem)

        def pull_wait(t, u, half_buf, sem):
            src = i_hbm.at[pl.ds(c * half + t * _SCH + u * _SUB, _SUB)]
            pltpu.make_async_copy(
                src, iv.at[pl.ds(half_buf * _SUB, _SUB)], sem).wait()

        def scan(u, bsel, off):
            for g in range(32):
                d = iv[pl.ds(bsel * _SUB + g * 16, 16)]
                mask = (d >= lo) & (d < hi)
                slots_v = plsc.cumsum(mask.astype(jnp.int32)) + (off - 1)
                plsc.store_scatter(dbuf, [slots_v], d - lo, mask=mask)
                pv = (u * _SUB + g * 16) + lanes
                plsc.store_scatter(pbuf, [slots_v], pv, mask=mask)
                cnt = plsc.all_reduce_population_count(mask)
                off = off + cnt[0]

            @pl.when(off >= 512)
            def _():
                pltpu.async_copy(
                    hsh.at[pbuf.at[pl.ds(0, 512)]],
                    hrows.at[pl.ds(0, 512)], sem_g).wait()
                rmw(32)
                for g in range(32):
                    dbuf[pl.ds(g * 16, 16)] = dbuf[pl.ds(512 + g * 16, 16)]
                    pbuf[pl.ds(g * 16, 16)] = pbuf[pl.ds(512 + g * 16, 16)]

            return jnp.where(off >= 512, off - 512, off)

        def chunk_body(t, _):
            plsc.subcore_barrier()

            @pl.when(s == 0)
            def _():
                pltpu.async_copy(
                    h_hbm.at[pl.ds(c * half + t * _SCH, _SCH)], hsh, sem_h)
                pltpu.make_async_copy(
                    h_hbm.at[pl.ds(c * half + t * _SCH, _SCH)], hsh,
                    sem_h).wait()

            pull(t, 0, 0, sem_a)
            plsc.subcore_barrier()

            def pair_body(b, off):
                u = 2 * b
                pull_wait(t, u, 0, sem_a)
                pull(t, u + 1, 1, sem_b)
                off = scan(u, 0, off)
                pull_wait(t, u + 1, 1, sem_b)
                pull(t, u + 2, 0, sem_a)
                off = scan(u + 1, 1, off)
                return off

            off = lax.fori_loop(0, (n_sub - 1) // 2, pair_body, 0)
            pull_wait(t, n_sub - 1, 0, sem_a)
            off = scan(n_sub - 1, 0, off)

            dbuf[pl.ds(off, 16)] = pad16
            dbuf[pl.ds(off + 16, 16)] = pad16
            pbuf[pl.ds(off, 16)] = zi16
            pbuf[pl.ds(off + 16, 16)] = zi16
            ng = (off + 31) >> 4

            @pl.when(ng * 16 <= 144)
            def _():
                pltpu.async_copy(
                    hsh.at[pbuf.at[pl.ds(0, 144)]],
                    hrows.at[pl.ds(0, 144)], sem_g).wait()

            @pl.when(ng * 16 > 144)
            def _():
                pltpu.async_copy(
                    hsh.at[pbuf.at[pl.ds(0, 544)]],
                    hrows.at[pl.ds(0, 544)], sem_g).wait()

            rmw(ng)
            return 0

        lax.fori_loop(0, n_chunks, chunk_body, 0)

        obase = c * n * 16 + s * _NPT * 16
        pltpu.sync_copy(acc.at[pl.ds(0, _NPT * 16)],
                        out_hbm.at[pl.ds(obase, _NPT * 16)])

    return body(h2, i_idx)


def kernel(x, edge_index, edge_attr, W1a, b1a, W2a, b2a, W1b, b1b, W2b, b2b, Wl1, bl1, Wl2, bl2):
    n = x.shape[0]
    e = edge_index.shape[1]
    i = edge_index[0]
    j = edge_index[1]

    r1p, r2p = _edge_pre(
        edge_attr.reshape(e // 8, 64),
        _kron8(W1a[:, 8:16].T), _tile8(b1a),
        _kron8(W1b[:, 32:40].T), _tile8(b1b),
    )

    p1p, q1p = _node_tables(
        x.reshape(n // 8, 32), _kron8(W1a[:, 0:4].T), _kron8(W1a[:, 4:8].T))
    pi1, qj1 = _sc_gather2(p1p.reshape(n, 16), q1p.reshape(n, 16), i, j)
    h2p = _edge_mlp(pi1.reshape(e // 8, 128), qj1.reshape(e // 8, 128), r1p,
                    _kron8(W2a.T), _tile8(b2a))
    part1 = _sc_scatter_max(h2p.reshape(e, 16), i, n).reshape(2, n // 8, 128)

    p2p, q2p = _node_tables_merge(
        part1[0], part1[1], _kron8(W1b[:, 0:16].T), _kron8(W1b[:, 16:32].T))
    pi2, qj2 = _sc_gather2(p2p.reshape(n, 16), q2p.reshape(n, 16), i, j)
    h2bp = _edge_mlp(pi2.reshape(e // 8, 128), qj2.reshape(e // 8, 128), r2p,
                     _kron8(W2b.T), _tile8(b2b))
    part2 = _sc_scatter_max(h2bp.reshape(e, 16), i, n).reshape(2, n // 8, 128)

    out = _head(part2[0], part2[1], _kron8(Wl1.T), _tile8(bl1),
                _kron8(Wl2.T), _tile8(bl2))
    return out.reshape(n, 1)


# speedup vs baseline: 8.5772x; 1.0039x over previous
import functools

import jax
import jax.numpy as jnp
from jax import lax
from jax.experimental import pallas as pl
from jax.experimental.pallas import tpu as pltpu
from jax.experimental.pallas import tpu_sc as plsc

EBLK = 12800
NBLK = 10000


EBLKP = 4000
NROWS = 12500


def _kron8(w):
    return jnp.kron(jnp.eye(8, dtype=w.dtype), w)


def _tile8(b):
    return jnp.tile(b, 8).reshape(1, -1)


def _tables_body(x_ref, wp_ref, wq_ref, p_ref, q_ref):
    x = x_ref[...]
    p_ref[...] = jnp.dot(x, wp_ref[...], preferred_element_type=jnp.float32)
    q_ref[...] = jnp.dot(x, wq_ref[...], preferred_element_type=jnp.float32)


def _node_tables(x_p, wp, wq):
    m, d = x_p.shape
    return pl.pallas_call(
        _tables_body,
        out_shape=[
            jax.ShapeDtypeStruct((m, 128), jnp.float32),
            jax.ShapeDtypeStruct((m, 128), jnp.float32),
        ],
    )(x_p, wp, wq)


def _tables_merge_body(xa_ref, xb_ref, wp_ref, wq_ref, p_ref, q_ref):
    x = jnp.maximum(xa_ref[...], xb_ref[...])
    p_ref[...] = jnp.dot(x, wp_ref[...], preferred_element_type=jnp.float32)
    q_ref[...] = jnp.dot(x, wq_ref[...], preferred_element_type=jnp.float32)


def _node_tables_merge(xa, xb, wp, wq):
    m = xa.shape[0]
    return pl.pallas_call(
        _tables_merge_body,
        out_shape=[
            jax.ShapeDtypeStruct((m, 128), jnp.float32),
            jax.ShapeDtypeStruct((m, 128), jnp.float32),
        ],
    )(xa, xb, wp, wq)


def _edge_pre_body(ea_ref, wa_ref, ba_ref, wb_ref, bb_ref, r1_ref, r2_ref):
    ea = ea_ref[...]
    r1_ref[...] = jnp.dot(ea, wa_ref[...], preferred_element_type=jnp.float32) + ba_ref[...]
    r2_ref[...] = jnp.dot(ea, wb_ref[...], preferred_element_type=jnp.float32) + bb_ref[...]


def _edge_pre(ea_p, wa, ba, wb, bb):
    m = ea_p.shape[0]
    grid = (m // EBLKP,)
    return pl.pallas_call(
        _edge_pre_body,
        grid=grid,
        in_specs=[
            pl.BlockSpec((EBLKP, 64), lambda b: (b, 0)),
            pl.BlockSpec((64, 128), lambda b: (0, 0)),
            pl.BlockSpec((1, 128), lambda b: (0, 0)),
            pl.BlockSpec((64, 128), lambda b: (0, 0)),
            pl.BlockSpec((1, 128), lambda b: (0, 0)),
        ],
        out_specs=[
            pl.BlockSpec((EBLKP, 128), lambda b: (b, 0)),
            pl.BlockSpec((EBLKP, 128), lambda b: (b, 0)),
        ],
        out_shape=[
            jax.ShapeDtypeStruct((m, 128), jnp.float32),
            jax.ShapeDtypeStruct((m, 128), jnp.float32),
        ],
    )(ea_p, wa, ba, wb, bb)


def _edge_mlp_body(pi_ref, qj_ref, r_ref, w2_ref, b2_ref, out_ref):
    h1 = jnp.maximum(pi_ref[...] + qj_ref[...] + r_ref[...], 0.0)
    h2 = jnp.dot(h1, w2_ref[...], preferred_element_type=jnp.float32) + b2_ref[...]
    out_ref[...] = jnp.maximum(h2, 0.0)


def _edge_mlp(pi, qj, r, w2, b2):
    m = pi.shape[0]
    grid = (m // EBLKP,)
    return pl.pallas_call(
        _edge_mlp_body,
        grid=grid,
        in_specs=[
            pl.BlockSpec((EBLKP, 128), lambda b: (b, 0)),
            pl.BlockSpec((EBLKP, 128), lambda b: (b, 0)),
            pl.BlockSpec((EBLKP, 128), lambda b: (b, 0)),
            pl.BlockSpec((128, 128), lambda b: (0, 0)),
            pl.BlockSpec((1, 128), lambda b: (0, 0)),
        ],
        out_specs=pl.BlockSpec((EBLKP, 128), lambda b: (b, 0)),
        out_shape=jax.ShapeDtypeStruct((m, 128), jnp.float32),
    )(pi, qj, r, w2, b2)


def _head_body(xa_ref, xb_ref, w1_ref, b1_ref, w2_ref, b2_ref, out_ref):
    x = jnp.maximum(xa_ref[...], xb_ref[...])
    h = jnp.maximum(jnp.dot(x, w1_ref[...], preferred_element_type=jnp.float32) + b1_ref[...], 0.0)
    out_ref[...] = jnp.dot(h, w2_ref[...], preferred_element_type=jnp.float32) + b2_ref[...]


def _head(xa, xb, wl1, bl1, wl2, bl2):
    m = xa.shape[0]
    return pl.pallas_call(
        _head_body,
        out_shape=jax.ShapeDtypeStruct((m, 8), jnp.float32),
    )(xa, xb, wl1, bl1, wl2, bl2)


_NC = 2
_NS = 16
_NW = _NC * _NS
_GCH = 1000


def _sc_gather2(p, q, i_idx, j_idx):
    e = i_idx.shape[0]
    per_w = e // _NW
    n_steps = per_w // _GCH
    n_pairs = n_steps // 2
    mesh = plsc.VectorSubcoreMesh(
        core_axis_name="c", subcore_axis_name="s", num_cores=_NC, num_subcores=_NS)

    @functools.partial(
        pl.kernel,
        out_type=[
            jax.ShapeDtypeStruct((e, 16), jnp.float32),
            jax.ShapeDtypeStruct((e, 16), jnp.float32),
        ],
        mesh=mesh,
        scratch_types=[
            pltpu.VMEM((2, _GCH), jnp.int32),
            pltpu.VMEM((2, _GCH), jnp.int32),
            pltpu.VMEM((2, _GCH, 16), jnp.float32),
            pltpu.VMEM((2, _GCH, 16), jnp.float32),
        ] + [pltpu.SemaphoreType.DMA] * 12,
        compiler_params=pltpu.CompilerParams(use_tc_tiling_on_sc=False),
    )
    def body(p_hbm, q_hbm, i_hbm, j_hbm, pi_hbm, qj_hbm, iv, jv, rp, rq,
             *sems):
        sii = sems[0:2]
        sjj = sems[2:4]
        sgp = sems[4:6]
        sgq = sems[6:8]
        sop = sems[8:10]
        soq = sems[10:12]
        wid = lax.axis_index("s") * _NC + lax.axis_index("c")
        base = wid * per_w

        def idx_copies(t, sl):
            off = base + t * _GCH
            return (
                (i_hbm.at[pl.ds(off, _GCH)], iv.at[sl], sii[sl]),
                (j_hbm.at[pl.ds(off, _GCH)], jv.at[sl], sjj[sl]),
            )

        def out_copies(t, sl):
            off = base + t * _GCH
            return (
                (rp.at[sl], pi_hbm.at[pl.ds(off, _GCH)], sop[sl]),
                (rq.at[sl], qj_hbm.at[pl.ds(off, _GCH)], soq[sl]),
            )

        def issue(copies):
            for src, dst, sem in copies:
                pltpu.async_copy(src, dst, sem)

        def wait(copies):
            for src, dst, sem in copies:
                pltpu.make_async_copy(src, dst, sem).wait()

        def gstep(t, sl, not_first, not_last):
            wait(idx_copies(t, sl))

            @pl.when(not_last)
            def _():
                issue(idx_copies(t + 1, sl ^ 1))

            @pl.when(not_first)
            def _():
                wait(out_copies(t - 2, sl))

            g1 = pltpu.async_copy(p_hbm.at[iv.at[sl]], rp.at[sl], sgp[sl])
            g2 = pltpu.async_copy(q_hbm.at[jv.at[sl]], rq.at[sl], sgq[sl])
            g1.wait()
            g2.wait()
            issue(out_copies(t, sl))

        issue(idx_copies(0, 0))

        def pair(b, carry):
            t0 = 2 * b
            gstep(t0, 0, b >= 1, t0 + 1 < n_steps)
            gstep(t0 + 1, 1, b >= 1, t0 + 2 < n_steps)
            return carry

        lax.fori_loop(0, n_pairs, pair, 0)
        wait(out_copies(n_steps - 2, 0))
        wait(out_copies(n_steps - 1, 1))

    return body(p, q, i_idx, j_idx)


_SCH = 12800
_SUB = 512
_NPT = 6250


def _sc_scatter_max(h2, i_idx, n):
    e = i_idx.shape[0]
    half = e // 2
    n_chunks = half // _SCH
    n_sub = _SCH // _SUB
    mesh = plsc.VectorSubcoreMesh(
        core_axis_name="c", subcore_axis_name="s", num_cores=_NC, num_subcores=_NS)

    @functools.partial(
        pl.kernel,
        out_type=jax.ShapeDtypeStruct((2 * n * 16,), jnp.float32),
        mesh=mesh,
        scratch_types=[
            pltpu.VMEM_SHARED((_SCH, 16), jnp.float32),
            pltpu.VMEM(((_NPT + 1) * 16,), jnp.float32),
            pltpu.VMEM((2 * _SUB,), jnp.int32),
            pltpu.VMEM((1040,), jnp.int32),
            pltpu.VMEM((1040,), jnp.int32),
            pltpu.VMEM((544, 16), jnp.float32),
            pltpu.SemaphoreType.DMA,
            pltpu.SemaphoreType.DMA,
            pltpu.SemaphoreType.DMA,
            pltpu.SemaphoreType.DMA,
        ],
        compiler_params=pltpu.CompilerParams(
            needs_layout_passes=False, use_tc_tiling_on_sc=False),
    )
    def body(h_hbm, i_hbm, out_hbm, hsh, acc, iv, dbuf, pbuf, hrows,
             sem_h, sem_g, sem_a, sem_b):
        c = lax.axis_index("c")
        s = lax.axis_index("s")
        lo = s * _NPT
        hi = lo + _NPT
        lanes = lax.iota(jnp.int32, 16)
        zeros16 = jnp.zeros((16,), jnp.float32)
        pad16 = jnp.full((16,), _NPT, jnp.int32)
        zi16 = jnp.zeros((16,), jnp.int32)

        def init_acc(g, _):
            acc[pl.ds(g * 16, 16)] = zeros16
            return 0
        lax.fori_loop(0, _NPT + 1, init_acc, 0)
        for g in range(65):
            pbuf[pl.ds(g * 16, 16)] = zi16
            dbuf[pl.ds(g * 16, 16)] = pad16

        def rmw(ng):
            def g_body(g, _):
                dv = dbuf[pl.ds(g * 16, 16)]
                for k in range(16):
                    dk = dv[k]
                    a = acc[pl.ds(dk * 16, 16)]
                    h = hrows[g * 16 + k, :]
                    acc[pl.ds(dk * 16, 16)] = jnp.maximum(a, h)
                return 0
            lax.fori_loop(0, ng, g_body, 0)

        def pull(t, u, half_buf, sem):
            src = i_hbm.at[pl.ds(c * half + t * _SCH + u * _SUB, _SUB)]
            pltpu.async_copy(src, iv.at[pl.ds(half_buf * _SUB, _SUB)], sem)

        def pull_wait(t, u, half_buf, sem):
            src = i_hbm.at[pl.ds(c * half + t * _SCH + u * _SUB, _SUB)]
            pltpu.make_async_copy(
                src, iv.at[pl.ds(half_buf * _SUB, _SUB)], sem).wait()

        def scan(u, bsel, off):
            for g in range(32):
                d = iv[pl.ds(bsel * _SUB + g * 16, 16)]
                mask = (d >= lo) & (d < hi)
                slots_v = plsc.cumsum(mask.astype(jnp.int32)) + (off - 1)
                plsc.store_scatter(dbuf, [slots_v], d - lo, mask=mask)
                pv = (u * _SUB + g * 16) + lanes
                plsc.store_scatter(pbuf, [slots_v], pv, mask=mask)
                cnt = plsc.all_reduce_population_count(mask)
                off = off + cnt[0]

            @pl.when(off >= 512)
            def _():
                pltpu.async_copy(
                    hsh.at[pbuf.at[pl.ds(0, 512)]],
                    hrows.at[pl.ds(0, 512)], sem_g).wait()
                rmw(32)
                for g in range(32):
                    dbuf[pl.ds(g * 16, 16)] = dbuf[pl.ds(512 + g * 16, 16)]
                    pbuf[pl.ds(g * 16, 16)] = pbuf[pl.ds(512 + g * 16, 16)]

            return jnp.where(off >= 512, off - 512, off)

        def chunk_body(t, _):
            plsc.subcore_barrier()

            @pl.when(s == 0)
            def _():
                pltpu.async_copy(
                    h_hbm.at[pl.ds(c * half + t * _SCH, _SCH)], hsh, sem_h)
                pltpu.make_async_copy(
                    h_hbm.at[pl.ds(c * half + t * _SCH, _SCH)], hsh,
                    sem_h).wait()

            pull(t, 0, 0, sem_a)
            plsc.subcore_barrier()

            def pair_body(b, off):
                u = 2 * b
                pull_wait(t, u, 0, sem_a)
                pull(t, u + 1, 1, sem_b)
                off = scan(u, 0, off)
                pull_wait(t, u + 1, 1, sem_b)
                pull(t, u + 2, 0, sem_a)
                off = scan(u + 1, 1, off)
                return off

            off = lax.fori_loop(0, (n_sub - 1) // 2, pair_body, 0)
            pull_wait(t, n_sub - 1, 0, sem_a)
            off = scan(n_sub - 1, 0, off)

            dbuf[pl.ds(off, 16)] = pad16
            dbuf[pl.ds(off + 16, 16)] = pad16
            pbuf[pl.ds(off, 16)] = zi16
            pbuf[pl.ds(off + 16, 16)] = zi16
            ng = (off + 31) >> 4

            @pl.when(ng * 16 <= 144)
            def _():
                pltpu.async_copy(
                    hsh.at[pbuf.at[pl.ds(0, 144)]],
                    hrows.at[pl.ds(0, 144)], sem_g).wait()

            @pl.when(ng * 16 > 144)
            def _():
                pltpu.async_copy(
                    hsh.at[pbuf.at[pl.ds(0, 544)]],
                    hrows.at[pl.ds(0, 544)], sem_g).wait()

            rmw(ng)
            return 0

        lax.fori_loop(0, n_chunks, chunk_body, 0)

        obase = c * n * 16 + s * _NPT * 16
        pltpu.sync_copy(acc.at[pl.ds(0, _NPT * 16)],
                        out_hbm.at[pl.ds(obase, _NPT * 16)])

    return body(h2, i_idx)


def kernel(x, edge_index, edge_attr, W1a, b1a, W2a, b2a, W1b, b1b, W2b, b2b, Wl1, bl1, Wl2, bl2):
    n = x.shape[0]
    e = edge_index.shape[1]
    i = edge_index[0]
    j = edge_index[1]

    r1p, r2p = _edge_pre(
        edge_attr.reshape(e // 8, 64),
        _kron8(W1a[:, 8:16].T), _tile8(b1a),
        _kron8(W1b[:, 32:40].T), _tile8(b1b),
    )

    p1p, q1p = _node_tables(
        x.reshape(n // 8, 32), _kron8(W1a[:, 0:4].T), _kron8(W1a[:, 4:8].T))
    pi1, qj1 = _sc_gather2(p1p.reshape(n, 16), q1p.reshape(n, 16), i, j)
    h2p = _edge_mlp(pi1.reshape(e // 8, 128), qj1.reshape(e // 8, 128), r1p,
                    _kron8(W2a.T), _tile8(b2a))
    part1 = _sc_scatter_max(h2p.reshape(e, 16), i, n).reshape(2, n // 8, 128)

    p2p, q2p = _node_tables_merge(
        part1[0], part1[1], _kron8(W1b[:, 0:16].T), _kron8(W1b[:, 16:32].T))
    pi2, qj2 = _sc_gather2(p2p.reshape(n, 16), q2p.reshape(n, 16), i, j)
    h2bp = _edge_mlp(pi2.reshape(e // 8, 128), qj2.reshape(e // 8, 128), r2p,
                     _kron8(W2b.T), _tile8(b2b))
    part2 = _sc_scatter_max(h2bp.reshape(e, 16), i, n).reshape(2, n // 8, 128)

    out = _head(part2[0], part2[1], _kron8(Wl1.T), _tile8(bl1),
                _kron8(Wl2.T), _tile8(bl2))
    return out.reshape(n, 1)
